# initial kernel scaffold (unmeasured)
import jax
import jax.numpy as jnp
from jax import lax
from jax.experimental import pallas as pl
from jax.experimental.pallas import tpu as pltpu


def kernel(
    x,
):
    def body(*refs):
        pass

    out_shape = jax.ShapeDtypeStruct(..., jnp.float32)
    return pl.pallas_call(body, out_shape=out_shape)(...)


# baseline (device time: 46039 ns/iter reference)
import jax
import jax.numpy as jnp
from jax import lax
from jax.experimental import pallas as pl
from jax.experimental.pallas import tpu as pltpu

N_Z = 4


def kernel(x):
    m_per, n = x.shape

    def body(x_ref, out_ref, comm_ref, send_sems, recv_sems):
        my_x = lax.axis_index("x")
        my_y = lax.axis_index("y")
        my_z = lax.axis_index("z")
        left = (my_z - 1) % N_Z
        right = (my_z + 1) % N_Z

        barrier_sem = pltpu.get_barrier_semaphore()
        for nbr in [left, right]:
            pl.semaphore_signal(
                barrier_sem, inc=1,
                device_id=(my_x, my_y, nbr),
                device_id_type=pl.DeviceIdType.MESH,
            )
        pl.semaphore_wait(barrier_sem, 2)

        out_ref[pl.ds(my_z * m_per, m_per), :] = x_ref[:, :]
        comm_ref[0, :, :] = x_ref[:, :]

        for h in range(N_Z - 1):
            send_slot = h % 2
            recv_slot = (h + 1) % 2
            rdma = pltpu.make_async_remote_copy(
                src_ref=comm_ref.at[send_slot],
                dst_ref=comm_ref.at[recv_slot],
                send_sem=send_sems.at[send_slot],
                recv_sem=recv_sems.at[recv_slot],
                device_id=(my_x, my_y, right),
                device_id_type=pl.DeviceIdType.MESH,
            )
            rdma.start()
            rdma.wait()

            origin = (my_z - h - 1) % N_Z
            out_ref[pl.ds(origin * m_per, m_per), :] = comm_ref[recv_slot, :, :]

    return pl.pallas_call(
        body,
        out_shape=jax.ShapeDtypeStruct((N_Z * m_per, n), x.dtype),
        in_specs=[pl.BlockSpec(memory_space=pltpu.VMEM)],
        out_specs=pl.BlockSpec(memory_space=pltpu.VMEM),
        scratch_shapes=[
            pltpu.VMEM((2, m_per, n), x.dtype),
            pltpu.SemaphoreType.DMA((2,)),
            pltpu.SemaphoreType.DMA((2,)),
        ],
        compiler_params=pltpu.CompilerParams(collective_id=0),
    )(x)


# device time: 37276 ns/iter; 1.2351x vs baseline; 1.2351x over previous
import jax
import jax.numpy as jnp
from jax import lax
from jax.experimental import pallas as pl
from jax.experimental.pallas import tpu as pltpu

N_Z = 4


def kernel(x):
    m_per, n = x.shape
    half = m_per // 2


    def body(x_ref, out_ref, send_up, recv_up, send_dn, recv_dn,
             send_x, recv_x):
        my_x = lax.axis_index("x")
        my_y = lax.axis_index("y")
        my_z = lax.axis_index("z")
        t = my_x
        has_up = my_z < N_Z - 1
        has_dn = my_z > 0

        def rows(o, tt):
            return o * m_per + tt * half

        def z_copy(o, dz, send_sems, recv_sems):
            return pltpu.make_async_remote_copy(
                src_ref=out_ref.at[pl.ds(rows(o, t), half), :],
                dst_ref=out_ref.at[pl.ds(rows(o, t), half), :],
                send_sem=send_sems.at[o],
                recv_sem=recv_sems.at[o],
                device_id=(my_x, my_y, my_z + dz),
                device_id_type=pl.DeviceIdType.MESH,
            )

        def x_copy(o):
            return pltpu.make_async_remote_copy(
                src_ref=out_ref.at[pl.ds(rows(o, t), half), :],
                dst_ref=out_ref.at[pl.ds(rows(o, t), half), :],
                send_sem=send_x.at[o],
                recv_sem=recv_x.at[o],
                device_id=(1 - my_x, my_y, my_z),
                device_id_type=pl.DeviceIdType.MESH,
            )

        def x_recv_desc(o):
            return pltpu.make_async_remote_copy(
                src_ref=out_ref.at[pl.ds(rows(o, 1 - t), half), :],
                dst_ref=out_ref.at[pl.ds(rows(o, 1 - t), half), :],
                send_sem=send_x.at[o],
                recv_sem=recv_x.at[o],
                device_id=(1 - my_x, my_y, my_z),
                device_id_type=pl.DeviceIdType.MESH,
            )

        def neighbor_signal(sem):
            pl.semaphore_signal(
                sem, inc=1, device_id=(1 - my_x, my_y, my_z),
                device_id_type=pl.DeviceIdType.MESH,
            )
            @pl.when(has_up)
            def _():
                pl.semaphore_signal(
                    sem, inc=1, device_id=(my_x, my_y, my_z + 1),
                    device_id_type=pl.DeviceIdType.MESH,
                )
            @pl.when(has_dn)
            def _():
                pl.semaphore_signal(
                    sem, inc=1, device_id=(my_x, my_y, my_z - 1),
                    device_id_type=pl.DeviceIdType.MESH,
                )

        def neighbor_wait(sem):
            is_middle = jnp.logical_and(has_up, has_dn)
            @pl.when(is_middle)
            def _():
                pl.semaphore_wait(sem, 3)
            @pl.when(jnp.logical_not(is_middle))
            def _():
                pl.semaphore_wait(sem, 2)

        barrier_sem = pltpu.get_barrier_semaphore()
        neighbor_signal(barrier_sem)
        neighbor_wait(barrier_sem)

        out_ref[pl.ds(my_z * m_per, m_per), :] = x_ref[:, :]

        for s in range(1, N_Z):
            o_us = my_z - s + 1
            o_ds = my_z + s - 1
            o_ur = my_z - s
            o_dr = my_z + s

            @pl.when(jnp.logical_and(has_up, o_us >= 0))
            def _():
                z_copy(o_us, 1, send_up, recv_up).start()

            @pl.when(jnp.logical_and(has_dn, o_ds <= N_Z - 1))
            def _():
                z_copy(o_ds, -1, send_dn, recv_dn).start()

            @pl.when(o_ur >= 0)
            def _():
                z_copy(o_ur, 1, send_up, recv_up).wait_recv()
                x_copy(o_ur).start()

            @pl.when(o_dr <= N_Z - 1)
            def _():
                z_copy(o_dr, -1, send_dn, recv_dn).wait_recv()
                x_copy(o_dr).start()

        for o in range(N_Z):
            @pl.when(o != my_z)
            def _():
                x_recv_desc(o).wait_recv()

        for s in range(1, N_Z):
            o_us = my_z - s + 1
            o_ds = my_z + s - 1

            @pl.when(jnp.logical_and(has_up, o_us >= 0))
            def _():
                z_copy(o_us, 1, send_up, recv_up).wait_send()

            @pl.when(jnp.logical_and(has_dn, o_ds <= N_Z - 1))
            def _():
                z_copy(o_ds, -1, send_dn, recv_dn).wait_send()

        for o in range(N_Z):
            @pl.when(o != my_z)
            def _():
                x_copy(o).wait_send()

        import functools
        @functools.partial(
            pl.run_scoped, second_barrier=pltpu.SemaphoreType.REGULAR
        )
        def _(second_barrier):
            neighbor_signal(second_barrier)
            neighbor_wait(second_barrier)

    return pl.pallas_call(
        body,
        out_shape=jax.ShapeDtypeStruct((N_Z * m_per, n), x.dtype),
        in_specs=[pl.BlockSpec(memory_space=pltpu.VMEM)],
        out_specs=pl.BlockSpec(memory_space=pltpu.VMEM),
        scratch_shapes=[
            pltpu.SemaphoreType.DMA((N_Z,)),
            pltpu.SemaphoreType.DMA((N_Z,)),
            pltpu.SemaphoreType.DMA((N_Z,)),
            pltpu.SemaphoreType.DMA((N_Z,)),
            pltpu.SemaphoreType.DMA((N_Z,)),
            pltpu.SemaphoreType.DMA((N_Z,)),
        ],
        compiler_params=pltpu.CompilerParams(collective_id=0),
    )(x)


# device time: 30652 ns/iter; 1.5020x vs baseline; 1.2161x over previous
import functools

import jax
import jax.numpy as jnp
from jax import lax
from jax.experimental import pallas as pl
from jax.experimental.pallas import tpu as pltpu

N_Z = 4


def kernel(x):
    m_per, n = x.shape
    half = m_per // 2
    quart = m_per // 4
    piece = m_per // 8

    def body(x_ref, out_ref,
             send_up, recv_up, send_dn, recv_dn,
             sx_own, rx_own, sy_own, ry_own,
             s_rel_x, r_diag_x, s_rel_y, r_diag_y):
        my_x = lax.axis_index("x")
        my_y = lax.axis_index("y")
        my_z = lax.axis_index("z")
        has_up = my_z < N_Z - 1
        has_dn = my_z > 0

        def qbase(o, xx, yy):
            return o * m_per + xx * half + yy * quart

        def copy(row0, nrows, ssem, rsem, o, dev):
            return pltpu.make_async_remote_copy(
                src_ref=out_ref.at[pl.ds(row0, nrows), :],
                dst_ref=out_ref.at[pl.ds(row0, nrows), :],
                send_sem=ssem.at[o],
                recv_sem=rsem.at[o],
                device_id=dev,
                device_id_type=pl.DeviceIdType.MESH,
            )

        def z_copy(o, dz, ssem, rsem):
            return copy(qbase(o, my_x, my_y), quart, ssem, rsem, o,
                        (my_x, my_y, my_z + dz))

        def x_own(o):
            return copy(qbase(o, my_x, my_y), quart, sx_own, rx_own, o,
                        (1 - my_x, my_y, my_z))

        def y_own(o):
            return copy(qbase(o, my_x, my_y), quart, sy_own, ry_own, o,
                        (my_x, 1 - my_y, my_z))

        def rel_x(o):
            return copy(qbase(o, my_x, 1 - my_y), piece, s_rel_x, r_diag_x,
                        o, (1 - my_x, my_y, my_z))

        def rel_y(o):
            return copy(qbase(o, 1 - my_x, my_y) + piece, piece, s_rel_y,
                        r_diag_y, o, (my_x, 1 - my_y, my_z))

        def diag_x_desc(o):
            return copy(qbase(o, 1 - my_x, 1 - my_y), piece, s_rel_x,
                        r_diag_x, o, (1 - my_x, my_y, my_z))

        def diag_y_desc(o):
            return copy(qbase(o, 1 - my_x, 1 - my_y) + piece, piece,
                        s_rel_y, r_diag_y, o, (my_x, 1 - my_y, my_z))

        def neighbor_signal(sem):
            for dev in ((1 - my_x, my_y, my_z), (my_x, 1 - my_y, my_z)):
                pl.semaphore_signal(
                    sem, inc=1, device_id=dev,
                    device_id_type=pl.DeviceIdType.MESH,
                )
            @pl.when(has_up)
            def _():
                pl.semaphore_signal(
                    sem, inc=1, device_id=(my_x, my_y, my_z + 1),
                    device_id_type=pl.DeviceIdType.MESH,
                )
            @pl.when(has_dn)
            def _():
                pl.semaphore_signal(
                    sem, inc=1, device_id=(my_x, my_y, my_z - 1),
                    device_id_type=pl.DeviceIdType.MESH,
                )

        def neighbor_wait(sem):
            is_middle = jnp.logical_and(has_up, has_dn)
            @pl.when(is_middle)
            def _():
                pl.semaphore_wait(sem, 4)
            @pl.when(jnp.logical_not(is_middle))
            def _():
                pl.semaphore_wait(sem, 3)

        barrier_sem = pltpu.get_barrier_semaphore()
        neighbor_signal(barrier_sem)
        neighbor_wait(barrier_sem)

        out_ref[pl.ds(my_z * m_per, m_per), :] = x_ref[:, :]

        for s in range(1, N_Z):
            o_us = my_z - s + 1
            o_ds = my_z + s - 1
            o_ur = my_z - s
            o_dr = my_z + s

            @pl.when(jnp.logical_and(has_up, o_us >= 0))
            def _():
                z_copy(o_us, 1, send_up, recv_up).start()

            @pl.when(jnp.logical_and(has_dn, o_ds <= N_Z - 1))
            def _():
                z_copy(o_ds, -1, send_dn, recv_dn).start()

            @pl.when(o_ur >= 0)
            def _():
                z_copy(o_ur, 1, send_up, recv_up).wait_recv()
                x_own(o_ur).start()
                y_own(o_ur).start()

            @pl.when(o_dr <= N_Z - 1)
            def _():
                z_copy(o_dr, -1, send_dn, recv_dn).wait_recv()
                x_own(o_dr).start()
                y_own(o_dr).start()

        for s in range(1, N_Z):
            for o in (my_z - s, my_z + s):
                @pl.when(jnp.logical_and(o >= 0, o <= N_Z - 1))
                def _():
                    y_own(o).wait_recv()
                    rel_x(o).start()
                    x_own(o).wait_recv()
                    rel_y(o).start()

        for s in range(1, N_Z):
            for o in (my_z - s, my_z + s):
                @pl.when(jnp.logical_and(o >= 0, o <= N_Z - 1))
                def _():
                    diag_x_desc(o).wait_recv()
                    diag_y_desc(o).wait_recv()

        for s in range(1, N_Z):
            o_us = my_z - s + 1
            o_ds = my_z + s - 1

            @pl.when(jnp.logical_and(has_up, o_us >= 0))
            def _():
                z_copy(o_us, 1, send_up, recv_up).wait_send()

            @pl.when(jnp.logical_and(has_dn, o_ds <= N_Z - 1))
            def _():
                z_copy(o_ds, -1, send_dn, recv_dn).wait_send()

        for o in range(N_Z):
            @pl.when(o != my_z)
            def _():
                x_own(o).wait_send()
                y_own(o).wait_send()
                rel_x(o).wait_send()
                rel_y(o).wait_send()

        @functools.partial(
            pl.run_scoped, second_barrier=pltpu.SemaphoreType.REGULAR
        )
        def _(second_barrier):
            neighbor_signal(second_barrier)
            neighbor_wait(second_barrier)

    dma = pltpu.SemaphoreType.DMA((N_Z,))
    return pl.pallas_call(
        body,
        out_shape=jax.ShapeDtypeStruct((N_Z * m_per, n), x.dtype),
        in_specs=[pl.BlockSpec(memory_space=pltpu.VMEM)],
        out_specs=pl.BlockSpec(memory_space=pltpu.VMEM),
        scratch_shapes=[dma] * 12,
        compiler_params=pltpu.CompilerParams(collective_id=0),
    )(x)
